# baseline (speedup 1.0000x reference)
import jax
import jax.numpy as jnp
from jax import lax
from jax.experimental import pallas as pl
from jax.experimental.pallas import tpu as pltpu

_BB = 16


def _stn_call(x_hi, x_lo, q_hmaj, w1c_hi, w1c_lo, b1, w2c, b2,
              w1fc, b1fc, w2fc, b2fc, e1, e2, dims):
    (B, Cin, H, W, k1, k2, Cout1, Cout2, Cq, Hq, Wq, out_h, out_w) = dims
    f32 = jnp.float32
    bf16 = jnp.bfloat16

    Ho1, Wo1 = H - k1 + 1, W - k1 + 1
    Hp1, Wp1 = Ho1 // 2, Wo1 // 2
    Ho2, Wo2 = Hp1 - k2 + 1, Wp1 - k2 + 1
    Hp2, Wp2 = Ho2 // 2, Wo2 // 2

    HWp = H * W + (k1 - 1)
    N1 = Ho1 * W
    N2 = Ho2 * Wp1
    P1 = Hp1 * Wp1
    P1_pad = P1 + (k2 - 1)
    P2 = Hp2 * Wp2
    C2p = 16
    n_hidden = b1fc.shape[1]
    n_out = b2fc.shape[1]
    M = out_h * out_w
    BB = _BB
    ow1 = float(max(out_w - 1, 1))
    oh1 = float(max(out_h - 1, 1))
    wq1 = float(Wq - 1)
    hq1 = float(Hq - 1)

    def _body(xh_ref, xl_ref, q_ref, w1h_ref, w1l_ref, b1_ref, w2c_ref,
              b2_ref, w1fc_ref, b1fc_ref, w2fc_ref, b2fc_ref, e1_ref,
              e2_ref, o_ref, p1_ref, p2_ref):
        lin = lax.broadcasted_iota(jnp.int32, (1, M), 1)
        yi = lin // out_w
        xi = lin - yi * out_w
        xn = 2.0 * xi.astype(f32) / ow1 - 1.0
        yn = 2.0 * yi.astype(f32) / oh1 - 1.0
        wcolf = lax.broadcasted_iota(jnp.int32, (Wq, M), 0).astype(f32)
        hrowf = lax.broadcasted_iota(jnp.int32, (Hq, M), 0).astype(f32)

        a1s = []
        for b in range(BB):
            xh = xh_ref[b]
            xl = xl_ref[b]
            xch = jnp.concatenate(
                [xh[:, i * W + j:i * W + j + N1]
                 for i in range(k1) for j in range(k1)], axis=0)
            xcl = jnp.concatenate(
                [xl[:, i * W + j:i * W + j + N1]
                 for i in range(k1) for j in range(k1)], axis=0)
            a1 = (jnp.dot(w1h_ref[...], xch, preferred_element_type=f32)
                  + jnp.dot(w1h_ref[...], xcl, preferred_element_type=f32)
                  + jnp.dot(w1l_ref[...], xch, preferred_element_type=f32))
            a1s.append(jnp.maximum(a1 + b1_ref[...], 0.0))

        for b in range(BB):
            a1 = a1s[b]
            vm = jnp.maximum(a1[:, :N1 - W], a1[:, W:])
            hm = jnp.maximum(vm[:, :N1 - W - 1], vm[:, 1:])
            p1_ref[b, :, P1:] = jnp.zeros((Cout1, P1_pad - P1), f32)
            for yp in range(Hp1):
                seg = hm[:, 2 * yp * W:2 * yp * W + 2 * Wp1]
                p1_ref[b, :, yp * Wp1:(yp + 1) * Wp1] = jnp.dot(
                    seg, e1_ref[...], preferred_element_type=f32)

        hm2s = []
        for b in range(BB):
            p1v = p1_ref[b]
            xc2 = jnp.concatenate(
                [p1v[:, i * Wp1 + j:i * Wp1 + j + N2]
                 for i in range(k2) for j in range(k2)], axis=0)
            a2 = jnp.dot(w2c_ref[...], xc2, preferred_element_type=f32)
            a2 = jnp.maximum(a2 + b2_ref[...], 0.0)
            vm2 = jnp.maximum(a2[:, :N2 - Wp1], a2[:, Wp1:])
            hm2s.append(jnp.maximum(vm2[:, :N2 - Wp1 - 1], vm2[:, 1:]))

        for yp in range(Hp2):
            seg_all = jnp.concatenate(
                [hm2s[b][:, 2 * yp * Wp1:2 * yp * Wp1 + 2 * Wp2]
                 for b in range(BB)], axis=0)
            pooled = jnp.dot(seg_all, e2_ref[...],
                             preferred_element_type=f32)
            p2_ref[:, :, yp * Wp2:(yp + 1) * Wp2] = pooled.reshape(BB, C2p, Wp2)

        h = b1fc_ref[...]
        for c in range(Cout2):
            h = h + jnp.dot(p2_ref[:, c, :], w1fc_ref[c],
                            preferred_element_type=f32)
        h = jnp.maximum(h, 0.0)
        th_all = jnp.dot(h, w2fc_ref[...], preferred_element_type=f32) \
            + b2fc_ref[...]

        sxs = []
        sys_ = []
        for b in range(BB):
            th = th_all[b:b + 1, :]
            gx = th[:, 0:1] * xn + th[:, 1:2] * yn + th[:, 2:3]
            gy = th[:, 3:4] * xn + th[:, 4:5] * yn + th[:, 5:6]
            ix = (gx + 1.0) * 0.5 * wq1
            iy = (gy + 1.0) * 0.5 * hq1
            sxs.append(jnp.maximum(1.0 - jnp.abs(wcolf - ix), 0.0).astype(bf16))
            sys_.append(jnp.maximum(1.0 - jnp.abs(hrowf - iy), 0.0))

        tmats = [jnp.dot(q_ref[b], sxs[b], preferred_element_type=f32)
                 for b in range(BB)]
        for b in range(BB):
            tmat, sy = tmats[b], sys_[b]
            acc = tmat[0:Cq, :] * sy[0:1, :]
            for hh in range(1, Hq):
                acc = acc + tmat[hh * Cq:(hh + 1) * Cq, :] * sy[hh:hh + 1, :]
            o_ref[b] = acc

    out = pl.pallas_call(
        _body,
        out_shape=jax.ShapeDtypeStruct((B, Cq, M), f32),
        grid=(B // BB,),
        in_specs=[
            pl.BlockSpec((BB, Cin, HWp), lambda n: (n, 0, 0)),
            pl.BlockSpec((BB, Cin, HWp), lambda n: (n, 0, 0)),
            pl.BlockSpec((BB, Hq * Cq, Wq), lambda n: (n, 0, 0)),
            pl.BlockSpec((Cout1, k1 * k1 * Cin), lambda n: (0, 0)),
            pl.BlockSpec((Cout1, k1 * k1 * Cin), lambda n: (0, 0)),
            pl.BlockSpec((Cout1, 1), lambda n: (0, 0)),
            pl.BlockSpec((C2p, k2 * k2 * Cout1), lambda n: (0, 0)),
            pl.BlockSpec((C2p, 1), lambda n: (0, 0)),
            pl.BlockSpec((Cout2, P2, n_hidden), lambda n: (0, 0, 0)),
            pl.BlockSpec((1, n_hidden), lambda n: (0, 0)),
            pl.BlockSpec((n_hidden, n_out), lambda n: (0, 0)),
            pl.BlockSpec((1, n_out), lambda n: (0, 0)),
            pl.BlockSpec((2 * Wp1, Wp1), lambda n: (0, 0)),
            pl.BlockSpec((2 * Wp2, Wp2), lambda n: (0, 0)),
        ],
        out_specs=pl.BlockSpec((BB, Cq, M), lambda n: (n, 0, 0)),
        scratch_shapes=[pltpu.VMEM((BB, Cout1, P1_pad), f32),
                        pltpu.VMEM((BB, C2p, P2), f32)],
        compiler_params=pltpu.CompilerParams(
            dimension_semantics=("parallel",)),
    )(x_hi, x_lo, q_hmaj, w1c_hi, w1c_lo, b1, w2c, b2,
      w1fc, b1fc, w2fc, b2fc, e1, e2)
    return out


@jax.jit
def _stn(w1, b1, w2, b2, w1fc, b1fc, w2fc, b2fc, x_nchw, query_nchw):
    B, Cin, H, W = x_nchw.shape
    _, Cq, Hq, Wq = query_nchw.shape
    t1, Cout1, _ = w1.shape
    t2, Cout2, _ = w2.shape
    k1 = int(round(t1 ** 0.5))
    k2 = int(round(t2 ** 0.5))
    Wp1 = (W - k1 + 1) // 2
    Wp2 = (Wp1 - k2 + 1) // 2
    f32 = jnp.float32
    bf16 = jnp.bfloat16
    C2p = 16

    x_flat = jnp.pad(x_nchw.reshape(B, Cin, H * W), ((0, 0), (0, 0), (0, k1 - 1)))
    x_hi = x_flat.astype(bf16)
    x_lo = (x_flat - x_hi.astype(f32)).astype(bf16)
    q_hmaj = jnp.transpose(query_nchw, (0, 2, 1, 3)).reshape(B, Hq * Cq, Wq)
    q_hmaj = q_hmaj.astype(bf16)
    w1c = jnp.transpose(w1, (1, 0, 2)).reshape(Cout1, t1 * Cin)
    w1c_hi = w1c.astype(bf16)
    w1c_lo = (w1c - w1c_hi.astype(f32)).astype(bf16)
    w2c = jnp.transpose(w2, (1, 0, 2)).reshape(Cout2, t2 * Cout1)
    w2c = jnp.pad(w2c, ((0, C2p - Cout2), (0, 0)))
    b2p = jnp.pad(b2, ((0, C2p - Cout2), (0, 0)))
    e1 = (jnp.arange(2 * Wp1)[:, None] == 2 * jnp.arange(Wp1)[None, :]).astype(f32)
    e2 = (jnp.arange(2 * Wp2)[:, None] == 2 * jnp.arange(Wp2)[None, :]).astype(f32)

    dims = (B, Cin, H, W, k1, k2, Cout1, Cout2, Cq, Hq, Wq, H, W)
    out = _stn_call(x_hi, x_lo, q_hmaj, w1c_hi, w1c_lo, b1, w2c, b2p,
                    w1fc, b1fc, w2fc, b2fc, e1, e2, dims)
    return out.reshape(B, Cq, H, W)


def kernel(w1, b1, w2, b2, w1fc, b1fc, w2fc, b2fc, x_nchw, query_nchw):
    return _stn(w1, b1, w2, b2, w1fc, b1fc, w2fc, b2fc, x_nchw, query_nchw)


# speedup vs baseline: 1.3121x; 1.3121x over previous
import jax
import jax.numpy as jnp
from jax import lax
from jax.experimental import pallas as pl
from jax.experimental.pallas import tpu as pltpu

_BB = 16


def _stn_call(x_flat, q_hmaj, w1c, b1, w2c, b2,
              w1fc, b1fc, w2fc, b2fc, e1, e2, dims):
    (B, Cin, H, W, k1, k2, Cout1, Cout2, Cq, Hq, Wq, out_h, out_w) = dims
    f32 = jnp.float32
    bf16 = jnp.bfloat16

    Ho1, Wo1 = H - k1 + 1, W - k1 + 1
    Hp1, Wp1 = Ho1 // 2, Wo1 // 2
    Ho2, Wo2 = Hp1 - k2 + 1, Wp1 - k2 + 1
    Hp2, Wp2 = Ho2 // 2, Wo2 // 2

    HWp = H * W + (k1 - 1)
    N1 = Ho1 * W
    N2 = Ho2 * Wp1
    P1 = Hp1 * Wp1
    P1_pad = P1 + (k2 - 1)
    P2 = Hp2 * Wp2
    C2p = 16
    n_hidden = b1fc.shape[1]
    n_out = b2fc.shape[1]
    M = out_h * out_w
    BB = _BB
    ow1 = float(max(out_w - 1, 1))
    oh1 = float(max(out_h - 1, 1))
    wq1 = float(Wq - 1)
    hq1 = float(Hq - 1)

    def _body(x_ref, q_ref, w1c_ref, b1_ref, w2c_ref,
              b2_ref, w1fc_ref, b1fc_ref, w2fc_ref, b2fc_ref, e1_ref,
              e2_ref, o_ref, p1_ref, p2_ref):
        lin = lax.broadcasted_iota(jnp.int32, (1, M), 1)
        yi = lin // out_w
        xi = lin - yi * out_w
        xn = 2.0 * xi.astype(f32) / ow1 - 1.0
        yn = 2.0 * yi.astype(f32) / oh1 - 1.0
        wcolf = lax.broadcasted_iota(jnp.int32, (Wq, M), 0).astype(f32)
        hrowf = lax.broadcasted_iota(jnp.int32, (Hq, M), 0).astype(f32)

        a1s = []
        for b in range(BB):
            xs = x_ref[b]
            xc = jnp.concatenate(
                [xs[:, i * W + j:i * W + j + N1]
                 for i in range(k1) for j in range(k1)], axis=0)
            a1 = jnp.dot(w1c_ref[...], xc, preferred_element_type=f32)
            a1s.append(jnp.maximum(a1 + b1_ref[...], 0.0))

        for b in range(BB):
            a1 = a1s[b]
            vm = jnp.maximum(a1[:, :N1 - W], a1[:, W:])
            hm = jnp.maximum(vm[:, :N1 - W - 1], vm[:, 1:])
            p1_ref[b, :, P1:] = jnp.zeros((Cout1, P1_pad - P1), f32)
            for yp in range(Hp1):
                seg = hm[:, 2 * yp * W:2 * yp * W + 2 * Wp1]
                p1_ref[b, :, yp * Wp1:(yp + 1) * Wp1] = jnp.dot(
                    seg, e1_ref[...], preferred_element_type=f32)

        hm2s = []
        for b in range(BB):
            p1v = p1_ref[b]
            xc2 = jnp.concatenate(
                [p1v[:, i * Wp1 + j:i * Wp1 + j + N2]
                 for i in range(k2) for j in range(k2)], axis=0)
            a2 = jnp.dot(w2c_ref[...], xc2, preferred_element_type=f32)
            a2 = jnp.maximum(a2 + b2_ref[...], 0.0)
            vm2 = jnp.maximum(a2[:, :N2 - Wp1], a2[:, Wp1:])
            hm2s.append(jnp.maximum(vm2[:, :N2 - Wp1 - 1], vm2[:, 1:]))

        for yp in range(Hp2):
            seg_all = jnp.concatenate(
                [hm2s[b][:, 2 * yp * Wp1:2 * yp * Wp1 + 2 * Wp2]
                 for b in range(BB)], axis=0)
            pooled = jnp.dot(seg_all, e2_ref[...],
                             preferred_element_type=f32)
            p2_ref[:, :, yp * Wp2:(yp + 1) * Wp2] = pooled.reshape(BB, C2p, Wp2)

        h = b1fc_ref[...]
        for c in range(Cout2):
            h = h + jnp.dot(p2_ref[:, c, :], w1fc_ref[c],
                            preferred_element_type=f32)
        h = jnp.maximum(h, 0.0)
        th_all = jnp.dot(h, w2fc_ref[...], preferred_element_type=f32) \
            + b2fc_ref[...]

        sxs = []
        sys_ = []
        for b in range(BB):
            th = th_all[b:b + 1, :]
            gx = th[:, 0:1] * xn + th[:, 1:2] * yn + th[:, 2:3]
            gy = th[:, 3:4] * xn + th[:, 4:5] * yn + th[:, 5:6]
            ix = (gx + 1.0) * 0.5 * wq1
            iy = (gy + 1.0) * 0.5 * hq1
            sxs.append(jnp.maximum(1.0 - jnp.abs(wcolf - ix), 0.0).astype(bf16))
            sys_.append(jnp.maximum(1.0 - jnp.abs(hrowf - iy), 0.0))

        tmats = [jnp.dot(q_ref[b], sxs[b], preferred_element_type=f32)
                 for b in range(BB)]
        for b in range(BB):
            tmat, sy = tmats[b], sys_[b]
            acc = tmat[0:Cq, :] * sy[0:1, :]
            for hh in range(1, Hq):
                acc = acc + tmat[hh * Cq:(hh + 1) * Cq, :] * sy[hh:hh + 1, :]
            o_ref[b] = acc

    out = pl.pallas_call(
        _body,
        out_shape=jax.ShapeDtypeStruct((B, Cq, M), f32),
        grid=(B // BB,),
        in_specs=[
            pl.BlockSpec((BB, Cin, HWp), lambda n: (n, 0, 0)),
            pl.BlockSpec((BB, Hq * Cq, Wq), lambda n: (n, 0, 0)),
            pl.BlockSpec((Cout1, k1 * k1 * Cin), lambda n: (0, 0)),
            pl.BlockSpec((Cout1, 1), lambda n: (0, 0)),
            pl.BlockSpec((C2p, k2 * k2 * Cout1), lambda n: (0, 0)),
            pl.BlockSpec((C2p, 1), lambda n: (0, 0)),
            pl.BlockSpec((Cout2, P2, n_hidden), lambda n: (0, 0, 0)),
            pl.BlockSpec((1, n_hidden), lambda n: (0, 0)),
            pl.BlockSpec((n_hidden, n_out), lambda n: (0, 0)),
            pl.BlockSpec((1, n_out), lambda n: (0, 0)),
            pl.BlockSpec((2 * Wp1, Wp1), lambda n: (0, 0)),
            pl.BlockSpec((2 * Wp2, Wp2), lambda n: (0, 0)),
        ],
        out_specs=pl.BlockSpec((BB, Cq, M), lambda n: (n, 0, 0)),
        scratch_shapes=[pltpu.VMEM((BB, Cout1, P1_pad), f32),
                        pltpu.VMEM((BB, C2p, P2), f32)],
        compiler_params=pltpu.CompilerParams(
            dimension_semantics=("parallel",)),
    )(x_flat, q_hmaj, w1c, b1, w2c, b2,
      w1fc, b1fc, w2fc, b2fc, e1, e2)
    return out


@jax.jit
def _stn(w1, b1, w2, b2, w1fc, b1fc, w2fc, b2fc, x_nchw, query_nchw):
    B, Cin, H, W = x_nchw.shape
    _, Cq, Hq, Wq = query_nchw.shape
    t1, Cout1, _ = w1.shape
    t2, Cout2, _ = w2.shape
    k1 = int(round(t1 ** 0.5))
    k2 = int(round(t2 ** 0.5))
    Wp1 = (W - k1 + 1) // 2
    Wp2 = (Wp1 - k2 + 1) // 2
    f32 = jnp.float32
    bf16 = jnp.bfloat16
    C2p = 16

    x_flat = jnp.pad(x_nchw.reshape(B, Cin, H * W), ((0, 0), (0, 0), (0, k1 - 1)))
    q_hmaj = jnp.transpose(query_nchw, (0, 2, 1, 3)).reshape(B, Hq * Cq, Wq)
    q_hmaj = q_hmaj.astype(bf16)
    w1c = jnp.transpose(w1, (1, 0, 2)).reshape(Cout1, t1 * Cin)
    w2c = jnp.transpose(w2, (1, 0, 2)).reshape(Cout2, t2 * Cout1)
    w2c = jnp.pad(w2c, ((0, C2p - Cout2), (0, 0)))
    b2p = jnp.pad(b2, ((0, C2p - Cout2), (0, 0)))
    e1 = (jnp.arange(2 * Wp1)[:, None] == 2 * jnp.arange(Wp1)[None, :]).astype(f32)
    e2 = (jnp.arange(2 * Wp2)[:, None] == 2 * jnp.arange(Wp2)[None, :]).astype(f32)

    dims = (B, Cin, H, W, k1, k2, Cout1, Cout2, Cq, Hq, Wq, H, W)
    out = _stn_call(x_flat, q_hmaj, w1c, b1, w2c, b2p,
                    w1fc, b1fc, w2fc, b2fc, e1, e2, dims)
    return out.reshape(B, Cq, H, W)


def kernel(w1, b1, w2, b2, w1fc, b1fc, w2fc, b2fc, x_nchw, query_nchw):
    return _stn(w1, b1, w2, b2, w1fc, b1fc, w2fc, b2fc, x_nchw, query_nchw)


# speedup vs baseline: 1.4029x; 1.0692x over previous
import jax
import jax.numpy as jnp
from jax import lax
from jax.experimental import pallas as pl
from jax.experimental.pallas import tpu as pltpu

_BB = 16


def _stn_call(x_flat, q_hmaj, w1c, b1, w2c, b2,
              w1fc, b1fc, w2fc, b2fc, e1, e2, dims):
    (B, Cin, H, W, k1, k2, Cout1, Cout2, Cq, Hq, Wq, out_h, out_w) = dims
    f32 = jnp.float32
    bf16 = jnp.bfloat16

    Ho1, Wo1 = H - k1 + 1, W - k1 + 1
    Hp1, Wp1 = Ho1 // 2, Wo1 // 2
    Ho2, Wo2 = Hp1 - k2 + 1, Wp1 - k2 + 1
    Hp2, Wp2 = Ho2 // 2, Wo2 // 2

    HWp = H * W + (k1 - 1)
    N1 = Ho1 * W
    N2 = Ho2 * Wp1
    P1 = Hp1 * Wp1
    P1_pad = P1 + (k2 - 1)
    P2 = Hp2 * Wp2
    C2p = 16
    n_hidden = b1fc.shape[1]
    n_out = b2fc.shape[1]
    M = out_h * out_w
    BB = _BB
    ow1 = float(max(out_w - 1, 1))
    oh1 = float(max(out_h - 1, 1))
    wq1 = float(Wq - 1)
    hq1 = float(Hq - 1)

    def _body(x_ref, q_ref, w1c_ref, b1_ref, w2c_ref,
              b2_ref, w1fc_ref, b1fc_ref, w2fc_ref, b2fc_ref, e1_ref,
              e2_ref, o_ref, p1_ref, p2_ref):
        lin = lax.broadcasted_iota(jnp.int32, (1, M), 1)
        yi = lin // out_w
        xi = lin - yi * out_w
        xn = 2.0 * xi.astype(f32) / ow1 - 1.0
        yn = 2.0 * yi.astype(f32) / oh1 - 1.0
        wcolf = lax.broadcasted_iota(jnp.int32, (Wq, M), 0).astype(f32)
        hrowf = lax.broadcasted_iota(jnp.int32, (Hq, M), 0).astype(f32)

        a1s = []
        for b in range(BB):
            xs = x_ref[b]
            xc = jnp.concatenate(
                [xs[:, i * W + j:i * W + j + N1]
                 for i in range(k1) for j in range(k1)], axis=0)
            a1 = jnp.dot(w1c_ref[...], xc, preferred_element_type=f32)
            a1s.append(jnp.maximum(a1 + b1_ref[...], 0.0))

        hms = []
        for b in range(BB):
            a1 = a1s[b]
            vm = jnp.maximum(a1[:, :N1 - W], a1[:, W:])
            hms.append(jnp.maximum(vm[:, :N1 - W - 1], vm[:, 1:]))
        p1_ref[:, :, P1:] = jnp.zeros((BB, Cout1, P1_pad - P1), f32)
        for yp in range(Hp1):
            seg_all = jnp.concatenate(
                [hms[b][:, 2 * yp * W:2 * yp * W + 2 * Wp1]
                 for b in range(BB)], axis=0)
            pooled = jnp.dot(seg_all, e1_ref[...],
                             preferred_element_type=f32)
            p1_ref[:, :, yp * Wp1:(yp + 1) * Wp1] = pooled.reshape(BB, Cout1, Wp1)

        hm2s = []
        for b in range(BB):
            p1v = p1_ref[b]
            xc2 = jnp.concatenate(
                [p1v[:, i * Wp1 + j:i * Wp1 + j + N2]
                 for i in range(k2) for j in range(k2)], axis=0)
            a2 = jnp.dot(w2c_ref[...], xc2, preferred_element_type=f32)
            a2 = jnp.maximum(a2 + b2_ref[...], 0.0)
            vm2 = jnp.maximum(a2[:, :N2 - Wp1], a2[:, Wp1:])
            hm2s.append(jnp.maximum(vm2[:, :N2 - Wp1 - 1], vm2[:, 1:]))

        for yp in range(Hp2):
            seg_all = jnp.concatenate(
                [hm2s[b][:, 2 * yp * Wp1:2 * yp * Wp1 + 2 * Wp2]
                 for b in range(BB)], axis=0)
            pooled = jnp.dot(seg_all, e2_ref[...],
                             preferred_element_type=f32)
            p2_ref[:, :, yp * Wp2:(yp + 1) * Wp2] = pooled.reshape(BB, C2p, Wp2)

        h = b1fc_ref[...]
        for c in range(Cout2):
            h = h + jnp.dot(p2_ref[:, c, :], w1fc_ref[c],
                            preferred_element_type=f32)
        h = jnp.maximum(h, 0.0)
        th_all = jnp.dot(h, w2fc_ref[...], preferred_element_type=f32) \
            + b2fc_ref[...]

        sxs = []
        sys_ = []
        for b in range(BB):
            th = th_all[b:b + 1, :]
            gx = th[:, 0:1] * xn + th[:, 1:2] * yn + th[:, 2:3]
            gy = th[:, 3:4] * xn + th[:, 4:5] * yn + th[:, 5:6]
            ix = (gx + 1.0) * 0.5 * wq1
            iy = (gy + 1.0) * 0.5 * hq1
            sxs.append(jnp.maximum(1.0 - jnp.abs(wcolf - ix), 0.0).astype(bf16))
            sys_.append(jnp.maximum(1.0 - jnp.abs(hrowf - iy), 0.0))

        tmats = [jnp.dot(q_ref[b], sxs[b], preferred_element_type=f32)
                 for b in range(BB)]
        for b in range(BB):
            tmat, sy = tmats[b], sys_[b]
            acc = tmat[0:Cq, :] * sy[0:1, :]
            for hh in range(1, Hq):
                acc = acc + tmat[hh * Cq:(hh + 1) * Cq, :] * sy[hh:hh + 1, :]
            o_ref[b] = acc

    out = pl.pallas_call(
        _body,
        out_shape=jax.ShapeDtypeStruct((B, Cq, M), f32),
        grid=(B // BB,),
        in_specs=[
            pl.BlockSpec((BB, Cin, HWp), lambda n: (n, 0, 0)),
            pl.BlockSpec((BB, Hq * Cq, Wq), lambda n: (n, 0, 0)),
            pl.BlockSpec((Cout1, k1 * k1 * Cin), lambda n: (0, 0)),
            pl.BlockSpec((Cout1, 1), lambda n: (0, 0)),
            pl.BlockSpec((C2p, k2 * k2 * Cout1), lambda n: (0, 0)),
            pl.BlockSpec((C2p, 1), lambda n: (0, 0)),
            pl.BlockSpec((Cout2, P2, n_hidden), lambda n: (0, 0, 0)),
            pl.BlockSpec((1, n_hidden), lambda n: (0, 0)),
            pl.BlockSpec((n_hidden, n_out), lambda n: (0, 0)),
            pl.BlockSpec((1, n_out), lambda n: (0, 0)),
            pl.BlockSpec((2 * Wp1, Wp1), lambda n: (0, 0)),
            pl.BlockSpec((2 * Wp2, Wp2), lambda n: (0, 0)),
        ],
        out_specs=pl.BlockSpec((BB, Cq, M), lambda n: (n, 0, 0)),
        scratch_shapes=[pltpu.VMEM((BB, Cout1, P1_pad), f32),
                        pltpu.VMEM((BB, C2p, P2), f32)],
        compiler_params=pltpu.CompilerParams(
            dimension_semantics=("parallel",)),
    )(x_flat, q_hmaj, w1c, b1, w2c, b2,
      w1fc, b1fc, w2fc, b2fc, e1, e2)
    return out


@jax.jit
def _stn(w1, b1, w2, b2, w1fc, b1fc, w2fc, b2fc, x_nchw, query_nchw):
    B, Cin, H, W = x_nchw.shape
    _, Cq, Hq, Wq = query_nchw.shape
    t1, Cout1, _ = w1.shape
    t2, Cout2, _ = w2.shape
    k1 = int(round(t1 ** 0.5))
    k2 = int(round(t2 ** 0.5))
    Wp1 = (W - k1 + 1) // 2
    Wp2 = (Wp1 - k2 + 1) // 2
    f32 = jnp.float32
    bf16 = jnp.bfloat16
    C2p = 16

    x_flat = jnp.pad(x_nchw.reshape(B, Cin, H * W), ((0, 0), (0, 0), (0, k1 - 1)))
    q_hmaj = jnp.transpose(query_nchw, (0, 2, 1, 3)).reshape(B, Hq * Cq, Wq)
    q_hmaj = q_hmaj.astype(bf16)
    w1c = jnp.transpose(w1, (1, 0, 2)).reshape(Cout1, t1 * Cin)
    w2c = jnp.transpose(w2, (1, 0, 2)).reshape(Cout2, t2 * Cout1)
    w2c = jnp.pad(w2c, ((0, C2p - Cout2), (0, 0)))
    b2p = jnp.pad(b2, ((0, C2p - Cout2), (0, 0)))
    e1 = (jnp.arange(2 * Wp1)[:, None] == 2 * jnp.arange(Wp1)[None, :]).astype(f32)
    e2 = (jnp.arange(2 * Wp2)[:, None] == 2 * jnp.arange(Wp2)[None, :]).astype(f32)

    dims = (B, Cin, H, W, k1, k2, Cout1, Cout2, Cq, Hq, Wq, H, W)
    out = _stn_call(x_flat, q_hmaj, w1c, b1, w2c, b2p,
                    w1fc, b1fc, w2fc, b2fc, e1, e2, dims)
    return out.reshape(B, Cq, H, W)


def kernel(w1, b1, w2, b2, w1fc, b1fc, w2fc, b2fc, x_nchw, query_nchw):
    return _stn(w1, b1, w2, b2, w1fc, b1fc, w2fc, b2fc, x_nchw, query_nchw)
